# Initial kernel scaffold; baseline (speedup 1.0000x reference)
import jax
import jax.numpy as jnp
from jax.experimental import pallas as pl


def kernel(input, pooling_mask):
    raise NotImplementedError("write your pallas kernel here")


# speedup vs baseline: 4.7204x; 4.7204x over previous
import functools

import jax
import jax.numpy as jnp
from jax import lax
from jax.experimental import pallas as pl
from jax.experimental.pallas import tpu as pltpu
from jax.experimental.pallas import tpu_sc as plsc

B = 8
C = 192
H = 224
W = 224
HW = H * W
K = HW // 4
NCHUNK = K // 16
NW = 32
CPB = NW // B
CPW = C // CPB


def _build():
    mesh = plsc.VectorSubcoreMesh(core_axis_name="c", subcore_axis_name="s")

    @functools.partial(
        pl.kernel,
        mesh=mesh,
        out_type=jax.ShapeDtypeStruct((B, C, K), jnp.float32),
        compiler_params=pltpu.CompilerParams(needs_layout_passes=False),
        scratch_types=[
            pltpu.VMEM((HW,), jnp.float32),
            pltpu.VMEM((K + 16,), jnp.int32),
            pltpu.VMEM((K,), jnp.float32),
        ],
    )
    def k(inp_hbm, mask_hbm, out_hbm, vbuf, idxbuf, obuf):
        cid = lax.axis_index("c")
        sid = lax.axis_index("s")
        wid = sid * 2 + cid
        b = wid // CPB
        ch0 = (wid % CPB) * CPW
        iota = lax.iota(jnp.int32, 16)

        pltpu.sync_copy(mask_hbm.at[b], vbuf)

        def zb(t, carry):
            idxbuf[pl.ds(t * 16, 16)] = jnp.zeros((16,), jnp.int32)
            return carry

        lax.fori_loop(0, NCHUNK + 1, zb, 0)

        def comp(rr, cnt):
            for t in range(W // 2 // 16):
                fpos = 2 * W * rr + 32 * t + 2 * iota
                mv = plsc.bitcast(plsc.load_gather(vbuf, [fpos]), jnp.int32)
                keep = mv >= 1
                k16 = keep.astype(jnp.int32)
                pos = cnt + plsc.cumsum(k16) - 1
                plsc.store_scatter(idxbuf, [pos], fpos, mask=keep)
                cnt = cnt + jnp.sum(k16)
            return cnt

        cnt = lax.fori_loop(0, H // 2, comp, jnp.int32(0))

        def chan(ci, carry):
            chv = ch0 + ci
            pltpu.sync_copy(inp_hbm.at[b * C + chv], vbuf)

            def g(j, c2):
                for u in range(4):
                    off = (j * 4 + u) * 16
                    v = idxbuf[pl.ds(off, 16)]
                    gath = plsc.load_gather(vbuf, [v])
                    sel = (off + iota) < cnt
                    obuf[pl.ds(off, 16)] = jnp.where(sel, gath, 0.0)
                return c2

            lax.fori_loop(0, NCHUNK // 4, g, 0)
            pltpu.sync_copy(obuf, out_hbm.at[b, chv])
            return carry

        lax.fori_loop(0, CPW, chan, 0)

    return k


def kernel(input, pooling_mask):
    inp_rows = input.reshape(B * C, HW)
    mask_rows = lax.bitcast_convert_type(
        pooling_mask.reshape(B, HW), jnp.float32)
    return _build()(inp_rows, mask_rows)


# speedup vs baseline: 13.0598x; 2.7667x over previous
import functools

import jax
import jax.numpy as jnp
from jax import lax
from jax.experimental import pallas as pl
from jax.experimental.pallas import tpu as pltpu
from jax.experimental.pallas import tpu_sc as plsc

B = 8
C = 192
H = 224
W = 224
HW = H * W
K = HW // 4
NCHUNK = K // 16
NW = 32
CPB = NW // B
CPW = C // CPB


def _build():
    mesh = plsc.VectorSubcoreMesh(core_axis_name="c", subcore_axis_name="s")

    @functools.partial(
        pl.kernel,
        mesh=mesh,
        out_type=jax.ShapeDtypeStruct((B, C, K), jnp.float32),
        compiler_params=pltpu.CompilerParams(needs_layout_passes=False),
        scratch_types=[
            pltpu.VMEM((HW,), jnp.float32),
            pltpu.VMEM((HW,), jnp.float32),
            pltpu.VMEM((K + 16,), jnp.int32),
            pltpu.VMEM((K,), jnp.float32),
            pltpu.SemaphoreType.DMA,
            pltpu.SemaphoreType.DMA,
        ],
    )
    def k(inp_hbm, mask_hbm, out_hbm, vbuf0, vbuf1, idxbuf, obuf, sem0, sem1):
        cid = lax.axis_index("c")
        sid = lax.axis_index("s")
        wid = sid * 2 + cid
        b = wid // CPB
        ch0 = (wid % CPB) * CPW
        row0 = b * C + ch0
        iota = lax.iota(jnp.int32, 16)

        pltpu.sync_copy(mask_hbm.at[b], vbuf0)

        @plsc.parallel_loop(0, NCHUNK + 1, unroll=4)
        def _(t):
            idxbuf[pl.ds(t * 16, 16)] = jnp.zeros((16,), jnp.int32)

        def comp(rr, cnt):
            for t in range(W // 2 // 16):
                fpos = 2 * W * rr + 32 * t + 2 * iota
                mv = plsc.bitcast(plsc.load_gather(vbuf0, [fpos]), jnp.int32)
                keep = mv >= 1
                k16 = keep.astype(jnp.int32)
                pos = cnt + plsc.cumsum(k16) - 1
                plsc.store_scatter(idxbuf, [pos], fpos, mask=keep)
                cnt = cnt + jnp.sum(k16)
            return cnt

        cnt = lax.fori_loop(0, H // 2, comp, jnp.int32(0))

        def gather_to_out(vbuf, chv):
            @plsc.parallel_loop(0, NCHUNK, unroll=8)
            def _(j):
                off = j * 16
                v = idxbuf[pl.ds(off, 16)]
                gath = plsc.load_gather(vbuf, [v])
                sel = (off + iota) < cnt
                obuf[pl.ds(off, 16)] = jnp.where(sel, gath, 0.0)

            pltpu.sync_copy(obuf, out_hbm.at[b, chv])

        last_row = B * C - 1
        pltpu.async_copy(inp_hbm.at[row0], vbuf0, sem0)

        def chan(i, carry):
            ra = row0 + 2 * i
            pltpu.make_async_copy(inp_hbm.at[ra], vbuf0, sem0).wait()
            pltpu.async_copy(
                inp_hbm.at[jnp.minimum(ra + 1, last_row)], vbuf1, sem1)
            gather_to_out(vbuf0, ch0 + 2 * i)
            pltpu.make_async_copy(inp_hbm.at[ra], vbuf1, sem1).wait()
            pltpu.async_copy(
                inp_hbm.at[jnp.minimum(ra + 2, last_row)], vbuf0, sem0)
            gather_to_out(vbuf1, ch0 + 2 * i + 1)
            return carry

        lax.fori_loop(0, CPW // 2, chan, 0)
        pltpu.make_async_copy(inp_hbm.at[row0], vbuf0, sem0).wait()

    return k


def kernel(input, pooling_mask):
    inp_rows = input.reshape(B * C, HW)
    mask_rows = lax.bitcast_convert_type(
        pooling_mask.reshape(B, HW), jnp.float32)
    return _build()(inp_rows, mask_rows)
